# baseline (speedup 1.0000x reference)
import functools

import jax
import jax.numpy as jnp
from jax import lax
from jax.experimental import pallas as pl
from jax.experimental.pallas import tpu as pltpu
from jax.experimental.pallas import tpu_sc as plsc

_NC = 2
_NS = 16
_LANE = 16
_TS = 8
_TL = 128


@functools.partial(jax.jit, static_argnums=(3, 4))
def _sc_native_gather(s_id, sidp, dbv, L, D):
    n_rows, _ = dbv.shape
    jblocks = n_rows // D
    B = s_id.shape[0]
    spw = B // _NS
    dh = D // _NC
    ngrp = dh // _LANE
    sch = _TL
    n_chunks = spw // sch

    mesh = plsc.VectorSubcoreMesh(core_axis_name="c", subcore_axis_name="s")

    @functools.partial(
        pl.kernel,
        mesh=mesh,
        out_type=jax.ShapeDtypeStruct((L, D, B), jnp.float32),
        compiler_params=pltpu.CompilerParams(needs_layout_passes=False),
        scratch_types=[
            pltpu.VMEM((spw,), jnp.int32),
            pltpu.VMEM((spw // _TL, _LANE), jnp.int32),
            pltpu.VMEM((dh,), jnp.int32),
            pltpu.VMEM((dh,), jnp.int32),
            pltpu.VMEM((dh,), jnp.int32),
            pltpu.VMEM((dh,), jnp.int32),
            pltpu.VMEM((2 * dh, _TL), jnp.float32),
            pltpu.VMEM((2 * dh, _TL), jnp.float32),
            pltpu.VMEM((_LANE,), jnp.int32),
            pltpu.VMEM((_LANE,), jnp.int32),
            pltpu.VMEM((L, dh, sch), jnp.float32),
            pltpu.SemaphoreType.DMA,
            pltpu.SemaphoreType.DMA,
            pltpu.SemaphoreType.DMA,
            pltpu.SemaphoreType.DMA,
        ],
    )
    def k(sid_hbm, sidp_hbm, db_hbm, out_hbm, sidv, sidpv, ix0, ix1,
          sx0, sx1, rb0, rb1, rz0, rz1, asm, sm0, sm1, st0, st1):
        h = lax.axis_index("c")
        g = lax.axis_index("s")
        base = g * spw
        d0 = h * dh
        pltpu.sync_copy(sid_hbm.at[pl.ds(base, spw)], sidv)
        pltpu.sync_copy(
            sidp_hbm.at[pl.ds(g * n_chunks, n_chunks)], sidpv
        )

        lanes = lax.iota(jnp.int32, _LANE)
        pgrp = []
        for u in range(ngrp):
            dv = lanes + (u * _LANE) + d0
            pgrp.append((dv // _TS) * (jblocks * _TS) + (dv % _TS))

        ixs = (ix0, ix1)
        sxs = (sx0, sx1)
        rbs = (rb0, rb1)
        rzs = (rz0, rz1)
        sms = (sm0, sm1)
        sts = (st0, st1)

        def stage(t, par, sv=None):
            if sv is None:
                sv = plsc.load_gather(
                    sidv, [jnp.zeros((_LANE,), jnp.int32) + t]
                )
            tv = sv * L
            jsv = lax.shift_right_logical(tv, 7)
            r0v = lax.bitwise_and(tv, _TL - 1)
            jnv = jnp.minimum(jsv + 1, jblocks - 1)
            for u in range(ngrp):
                ixs[par][pl.ds(u * _LANE, _LANE)] = pgrp[u] + jsv * _TS
                sxs[par][pl.ds(u * _LANE, _LANE)] = pgrp[u] + jnv * _TS
            rzs[par][...] = r0v
            straddle = lax.reduce_max(r0v, (0,)) > (_TL - L)
            pltpu.async_copy(
                db_hbm.at[ixs[par]], rbs[par].at[pl.ds(0, dh)], sms[par]
            )

            @pl.when(straddle)
            def _():
                pltpu.async_copy(
                    db_hbm.at[sxs[par]], rbs[par].at[pl.ds(dh, dh)], sts[par]
                )

            return straddle

        def drain(par, straddle):
            pltpu.make_async_copy(
                db_hbm.at[ixs[par]], rbs[par].at[pl.ds(0, dh)], sms[par]
            ).wait()

            @pl.when(straddle)
            def _():
                pltpu.make_async_copy(
                    db_hbm.at[sxs[par]], rbs[par].at[pl.ds(dh, dh)], sts[par]
                ).wait()

        rowbases = [lanes + (u * _LANE) for u in range(ngrp)]

        def extract(t, par):
            r0v = rzs[par][...]
            tsplat = jnp.zeros((_LANE,), jnp.int32) + t
            for l in range(L):
                cv = r0v + l
                sel = jnp.where(cv >= _TL, dh, 0)
                colv = lax.bitwise_and(cv, _TL - 1)
                lsplat = jnp.zeros((_LANE,), jnp.int32) + l
                for u in range(ngrp):
                    vals = plsc.load_gather(
                        rbs[par], [rowbases[u] + sel, colv]
                    )
                    plsc.store_scatter(
                        asm, [lsplat, rowbases[u], tsplat], vals
                    )

        for ch in range(n_chunks):
            c0 = ch * sch

            drain(0, stage(c0, 0, sv=sidpv[ch]))

            def body(t, carry):
                nxt = jnp.minimum(t + 1, sch - 1)
                p1 = stage(c0 + nxt, 1)
                extract(t, 0)
                drain(1, p1)
                p0 = stage(c0 + jnp.minimum(t + 2, sch - 1), 0)
                extract(t + 1, 1)
                drain(0, p0)
                return carry

            lax.fori_loop(0, sch // 2, lambda i, c: body(i * 2, c), 0)
            pltpu.sync_copy(
                asm, out_hbm.at[:, pl.ds(d0, dh), pl.ds(base + c0, sch)]
            )

    return k(s_id, sidp, dbv)


def kernel(s_id, sent, database):
    L = sent.shape[1]
    D = database.shape[1]
    NL = database.shape[0]
    sidp = jnp.broadcast_to(
        s_id[::_TL, None], (s_id.shape[0] // _TL, _LANE)
    )
    dbT = database.T
    y = dbT.reshape(D // _TS, _TS, NL // _TL, _TL)
    z = y.transpose(0, 2, 1, 3)
    dbv = z.reshape((D // _TS) * (NL // _TL) * _TS, _TL)
    outT = _sc_native_gather(s_id, sidp, dbv, L, D)
    return outT.transpose(2, 0, 1)


# speedup vs baseline: 1.0763x; 1.0763x over previous
import functools

import jax
import jax.numpy as jnp
from jax import lax
from jax.experimental import pallas as pl
from jax.experimental.pallas import tpu as pltpu
from jax.experimental.pallas import tpu_sc as plsc

_NC = 2
_NS = 16
_LANE = 16
_TS = 8
_TL = 128


@functools.partial(jax.jit, static_argnums=(3, 4))
def _sc_native_gather(s_id, sidp, dbv, L, D):
    n_rows, _ = dbv.shape
    jblocks = n_rows // D
    B = s_id.shape[0]
    spw = B // _NS
    dh = D // _NC
    ngrp = dh // _LANE
    sch = _TL
    n_chunks = spw // sch

    mesh = plsc.VectorSubcoreMesh(core_axis_name="c", subcore_axis_name="s")

    @functools.partial(
        pl.kernel,
        mesh=mesh,
        out_type=jax.ShapeDtypeStruct((L, D, B), jnp.float32),
        compiler_params=pltpu.CompilerParams(needs_layout_passes=False),
        scratch_types=[
            pltpu.VMEM((spw,), jnp.int32),
            pltpu.VMEM((2 * spw // _TL, _LANE), jnp.int32),
            [pltpu.VMEM((dh,), jnp.int32) for _ in range(4)],
            [pltpu.VMEM((dh,), jnp.int32) for _ in range(4)],
            [pltpu.VMEM((2 * dh, _TL), jnp.float32) for _ in range(4)],
            [pltpu.VMEM((_LANE,), jnp.int32) for _ in range(4)],
            pltpu.VMEM((L, dh, sch), jnp.float32),
            [pltpu.SemaphoreType.DMA for _ in range(4)],
            [pltpu.SemaphoreType.DMA for _ in range(4)],
        ],
    )
    def k(sid_hbm, sidp_hbm, db_hbm, out_hbm, sidv, sidpv, ixs,
          sxs, rbs, rzs, asm, sms, sts):
        h = lax.axis_index("c")
        g = lax.axis_index("s")
        base = g * spw
        d0 = h * dh
        pltpu.sync_copy(sid_hbm.at[pl.ds(base, spw)], sidv)
        pltpu.sync_copy(
            sidp_hbm.at[pl.ds(g * 2 * n_chunks, 2 * n_chunks)], sidpv
        )

        lanes = lax.iota(jnp.int32, _LANE)
        pgrp = []
        for u in range(ngrp):
            dv = lanes + (u * _LANE) + d0
            pgrp.append((dv // _TS) * (jblocks * _TS) + (dv % _TS))

        def stage(t, par, sv=None):
            if sv is None:
                sv = plsc.load_gather(
                    sidv, [jnp.zeros((_LANE,), jnp.int32) + t]
                )
            tv = sv * L
            jsv = lax.shift_right_logical(tv, 7)
            r0v = lax.bitwise_and(tv, _TL - 1)
            jnv = jnp.minimum(jsv + 1, jblocks - 1)
            for u in range(ngrp):
                ixs[par][pl.ds(u * _LANE, _LANE)] = pgrp[u] + jsv * _TS
                sxs[par][pl.ds(u * _LANE, _LANE)] = pgrp[u] + jnv * _TS
            rzs[par][...] = r0v
            straddle = lax.reduce_max(r0v, (0,)) > (_TL - L)
            pltpu.async_copy(
                db_hbm.at[ixs[par]], rbs[par].at[pl.ds(0, dh)], sms[par]
            )

            @pl.when(straddle)
            def _():
                pltpu.async_copy(
                    db_hbm.at[sxs[par]], rbs[par].at[pl.ds(dh, dh)], sts[par]
                )

        def drain(par):
            straddle = lax.reduce_max(rzs[par][...], (0,)) > (_TL - L)
            pltpu.make_async_copy(
                db_hbm.at[ixs[par]], rbs[par].at[pl.ds(0, dh)], sms[par]
            ).wait()

            @pl.when(straddle)
            def _():
                pltpu.make_async_copy(
                    db_hbm.at[sxs[par]], rbs[par].at[pl.ds(dh, dh)], sts[par]
                ).wait()

        rowbases = [lanes + (u * _LANE) for u in range(ngrp)]

        def extract(t, par):
            r0v = rzs[par][...]
            tsplat = jnp.zeros((_LANE,), jnp.int32) + t
            for l in range(L):
                cv = r0v + l
                sel = jnp.where(cv >= _TL, dh, 0)
                colv = lax.bitwise_and(cv, _TL - 1)
                lsplat = jnp.zeros((_LANE,), jnp.int32) + l
                for u in range(ngrp):
                    vals = plsc.load_gather(
                        rbs[par], [rowbases[u] + sel, colv]
                    )
                    plsc.store_scatter(
                        asm, [lsplat, rowbases[u], tsplat], vals
                    )

        for ch in range(n_chunks):
            c0 = ch * sch

            stage(c0, 0, sv=sidpv[2 * ch])
            stage(c0 + 1, 1, sv=sidpv[2 * ch + 1])

            def body(t, carry):
                for j in range(4):
                    stage(c0 + jnp.minimum(t + 2 + j, sch - 1), (j + 2) % 4)
                    drain(j)
                    extract(t + j, j)
                return carry

            lax.fori_loop(0, sch // 4, lambda i, c: body(i * 4, c), 0)
            drain(0)
            drain(1)
            pltpu.sync_copy(
                asm, out_hbm.at[:, pl.ds(d0, dh), pl.ds(base + c0, sch)]
            )

    return k(s_id, sidp, dbv)


def kernel(s_id, sent, database):
    L = sent.shape[1]
    D = database.shape[1]
    NL = database.shape[0]
    s2 = s_id.reshape(-1, _TL)[:, :2].reshape(-1)
    sidp = jnp.broadcast_to(s2[:, None], (s2.shape[0], _LANE))
    dbT = database.T
    y = dbT.reshape(D // _TS, _TS, NL // _TL, _TL)
    z = y.transpose(0, 2, 1, 3)
    dbv = z.reshape((D // _TS) * (NL // _TL) * _TS, _TL)
    outT = _sc_native_gather(s_id, sidp, dbv, L, D)
    return outT.transpose(2, 0, 1)


# speedup vs baseline: 1.0901x; 1.0128x over previous
import functools

import jax
import jax.numpy as jnp
from jax import lax
from jax.experimental import pallas as pl
from jax.experimental.pallas import tpu as pltpu
from jax.experimental.pallas import tpu_sc as plsc

_NC = 2
_NS = 16
_LANE = 16
_TS = 8
_TL = 128


@functools.partial(jax.jit, static_argnums=(3, 4))
def _sc_native_gather(s_id, sidp, dbv, L, D):
    n_rows, _ = dbv.shape
    jblocks = n_rows // D
    B = s_id.shape[0]
    spw = B // _NS
    dh = D // _NC
    ngrp = dh // _LANE
    sch = _TL
    n_chunks = spw // sch

    mesh = plsc.VectorSubcoreMesh(core_axis_name="c", subcore_axis_name="s")

    @functools.partial(
        pl.kernel,
        mesh=mesh,
        out_type=jax.ShapeDtypeStruct((L, D, B), jnp.float32),
        compiler_params=pltpu.CompilerParams(needs_layout_passes=False),
        scratch_types=[
            pltpu.VMEM((spw,), jnp.int32),
            pltpu.VMEM((2 * spw // _TL, _LANE), jnp.int32),
            [pltpu.VMEM((2 * dh,), jnp.int32) for _ in range(2)],
            [pltpu.VMEM((2 * dh,), jnp.int32) for _ in range(2)],
            [pltpu.VMEM((4 * dh, _TL), jnp.float32) for _ in range(2)],
            [pltpu.VMEM((2, _LANE), jnp.int32) for _ in range(2)],
            pltpu.VMEM((L, dh, sch), jnp.float32),
            [pltpu.SemaphoreType.DMA for _ in range(2)],
            [pltpu.SemaphoreType.DMA for _ in range(2)],
        ],
    )
    def k(sid_hbm, sidp_hbm, db_hbm, out_hbm, sidv, sidpv, ixs,
          sxs, rbs, rzs, asm, sms, sts):
        h = lax.axis_index("c")
        g = lax.axis_index("s")
        base = g * spw
        d0 = h * dh
        pltpu.sync_copy(sid_hbm.at[pl.ds(base, spw)], sidv)
        pltpu.sync_copy(
            sidp_hbm.at[pl.ds(g * 2 * n_chunks, 2 * n_chunks)], sidpv
        )

        lanes = lax.iota(jnp.int32, _LANE)
        pgrp = []
        for u in range(ngrp):
            dv = lanes + (u * _LANE) + d0
            pgrp.append((dv // _TS) * (jblocks * _TS) + (dv % _TS))

        def stage(t, par, sv0=None, sv1=None):
            if sv0 is None:
                sv0 = plsc.load_gather(
                    sidv, [jnp.zeros((_LANE,), jnp.int32) + t]
                )
                sv1 = plsc.load_gather(
                    sidv, [jnp.zeros((_LANE,), jnp.int32) + (t + 1)]
                )
            for j, sv in ((0, sv0), (1, sv1)):
                tv = sv * L
                jsv = lax.shift_right_logical(tv, 7)
                r0v = lax.bitwise_and(tv, _TL - 1)
                jnv = jnp.minimum(jsv + 1, jblocks - 1)
                for u in range(ngrp):
                    o = j * dh + u * _LANE
                    ixs[par][pl.ds(o, _LANE)] = pgrp[u] + jsv * _TS
                    sxs[par][pl.ds(o, _LANE)] = pgrp[u] + jnv * _TS
                rzs[par][j] = r0v
            straddle = (
                lax.reduce_max(jnp.maximum(rzs[par][0], rzs[par][1]), (0,))
                > (_TL - L)
            )
            pltpu.async_copy(
                db_hbm.at[ixs[par]], rbs[par].at[pl.ds(0, 2 * dh)], sms[par]
            )

            @pl.when(straddle)
            def _():
                pltpu.async_copy(
                    db_hbm.at[sxs[par]], rbs[par].at[pl.ds(2 * dh, 2 * dh)],
                    sts[par]
                )

        def drain(par):
            straddle = (
                lax.reduce_max(jnp.maximum(rzs[par][0], rzs[par][1]), (0,))
                > (_TL - L)
            )
            pltpu.make_async_copy(
                db_hbm.at[ixs[par]], rbs[par].at[pl.ds(0, 2 * dh)], sms[par]
            ).wait()

            @pl.when(straddle)
            def _():
                pltpu.make_async_copy(
                    db_hbm.at[sxs[par]], rbs[par].at[pl.ds(2 * dh, 2 * dh)],
                    sts[par]
                ).wait()

        rowbases = [lanes + (u * _LANE) for u in range(ngrp)]

        def extract(t, par, j):
            r0v = rzs[par][j]
            tsplat = jnp.zeros((_LANE,), jnp.int32) + t
            for l in range(L):
                cv = r0v + l
                sel = jnp.where(cv >= _TL, 2 * dh, 0) + (j * dh)
                colv = lax.bitwise_and(cv, _TL - 1)
                lsplat = jnp.zeros((_LANE,), jnp.int32) + l
                for u in range(ngrp):
                    vals = plsc.load_gather(
                        rbs[par], [rowbases[u] + sel, colv]
                    )
                    plsc.store_scatter(
                        asm, [lsplat, rowbases[u], tsplat], vals
                    )

        for ch in range(n_chunks):
            c0 = ch * sch

            stage(c0, 0, sv0=sidpv[2 * ch], sv1=sidpv[2 * ch + 1])

            def body(t, carry):
                stage(c0 + t + 2, 1)
                drain(0)
                extract(t, 0, 0)
                extract(t + 1, 0, 1)
                stage(c0 + jnp.minimum(t + 4, sch - 2), 0)
                drain(1)
                extract(t + 2, 1, 0)
                extract(t + 3, 1, 1)
                return carry

            lax.fori_loop(0, sch // 4, lambda i, c: body(i * 4, c), 0)
            drain(0)
            pltpu.sync_copy(
                asm, out_hbm.at[:, pl.ds(d0, dh), pl.ds(base + c0, sch)]
            )

    return k(s_id, sidp, dbv)


def kernel(s_id, sent, database):
    L = sent.shape[1]
    D = database.shape[1]
    NL = database.shape[0]
    s2 = s_id.reshape(-1, _TL)[:, :2].reshape(-1)
    sidp = jnp.broadcast_to(s2[:, None], (s2.shape[0], _LANE))
    dbT = database.T
    y = dbT.reshape(D // _TS, _TS, NL // _TL, _TL)
    z = y.transpose(0, 2, 1, 3)
    dbv = z.reshape((D // _TS) * (NL // _TL) * _TS, _TL)
    outT = _sc_native_gather(s_id, sidp, dbv, L, D)
    return outT.transpose(2, 0, 1)
